# Initial kernel scaffold; baseline (speedup 1.0000x reference)
import jax
import jax.numpy as jnp
from jax import lax
from jax.experimental import pallas as pl
from jax.experimental.pallas import tpu as pltpu
from jax.experimental.pallas import tpu_sc as plsc

_N = 10000
_E = 320000
_D = 128
_HF = 128
_C = 16
_B = 64

_NPAD = 10240
_NC = 2
_NS = 16
_NW = _NC * _NS
_EW = _E // _NW
_K = 125
_CH = _EW // _K
_RPS = _NPAD // _NS
_DEGW = 16
_BLK = 1024
_G = _NPAD // _BLK

_sc_mesh = plsc.VectorSubcoreMesh(core_axis_name="c", subcore_axis_name="s")


def _deg_body(dst3, ones_h, z16_h, out, dst_v, ones_v, acc):
    c = lax.axis_index("c")
    s = lax.axis_index("s")
    w = c * _NS + s
    pltpu.sync_copy(z16_h, acc.at[pl.ds(s * _RPS, _RPS)])
    pltpu.sync_copy(dst3.at[w], dst_v)
    pltpu.sync_copy(ones_h, ones_v)
    plsc.subcore_barrier()

    def body(j, carry):
        pltpu.sync_copy(ones_v, acc.at[dst_v.at[j]], add=True)
        return carry

    lax.fori_loop(0, _CH, body, 0)
    plsc.subcore_barrier()
    pltpu.sync_copy(acc.at[pl.ds(s * _RPS, _RPS)],
                    out.at[c].at[pl.ds(s * _RPS, _RPS)])


_deg_call = pl.kernel(
    _deg_body,
    out_type=jax.ShapeDtypeStruct((_NC, _NPAD, _DEGW), jnp.float32),
    mesh=_sc_mesh,
    scratch_types=[
        pltpu.VMEM((_CH, _K), jnp.int32),
        pltpu.VMEM((_K, _DEGW), jnp.float32),
        pltpu.VMEM_SHARED((_NPAD, _DEGW), jnp.float32),
    ],
)


def _scat_body(u_h, src3, dst3, z_h, out, src_v, dst_v, gbuf, acc):
    c = lax.axis_index("c")
    s = lax.axis_index("s")
    w = c * _NS + s
    pltpu.sync_copy(z_h, acc.at[pl.ds(s * _RPS, _RPS)])
    pltpu.sync_copy(src3.at[w], src_v)
    pltpu.sync_copy(dst3.at[w], dst_v)
    plsc.subcore_barrier()

    def body(j, carry):
        pltpu.sync_copy(u_h.at[src_v.at[j]], gbuf)
        pltpu.sync_copy(gbuf, acc.at[dst_v.at[j]], add=True)
        return carry

    lax.fori_loop(0, _CH, body, 0)
    plsc.subcore_barrier()
    pltpu.sync_copy(acc.at[pl.ds(s * _RPS, _RPS)],
                    out.at[c].at[pl.ds(s * _RPS, _RPS)])


_scat_call = pl.kernel(
    _scat_body,
    out_type=jax.ShapeDtypeStruct((_NC, _NPAD, _HF), jnp.float32),
    mesh=_sc_mesh,
    scratch_types=[
        pltpu.VMEM((_CH, _K), jnp.int32),
        pltpu.VMEM((_CH, _K), jnp.int32),
        pltpu.VMEM((_K, _HF), jnp.float32),
        pltpu.VMEM_SHARED((_NPAD, _HF), jnp.float32),
    ],
)


def _tc1_body(x_ref, w1_ref, degp_ref, u1_ref, dis_ref):
    deg = degp_ref[0, :, 0:1] + degp_ref[1, :, 0:1] + 1.0
    dis = lax.rsqrt(deg)
    h = jnp.dot(x_ref[...], w1_ref[...], preferred_element_type=jnp.float32)
    u1_ref[...] = dis * h
    dis_ref[...] = dis


def _mid_body(p_ref, u_ref, dis_ref, b_ref, w_ref, out_ref):
    dis = dis_ref[...]
    t = dis * (p_ref[0] + p_ref[1] + u_ref[...]) + b_ref[...]
    h = jnp.maximum(t, 0.0)
    out_ref[...] = dis * jnp.dot(h, w_ref[...],
                                 preferred_element_type=jnp.float32)


def _fin_body(p_ref, u_ref, dis_ref, b_ref, batch_ref, wl_ref, bl_ref,
              out_ref, seg, cnt):
    i = pl.program_id(0)

    @pl.when(i == 0)
    def _():
        seg[...] = jnp.zeros_like(seg)
        cnt[...] = jnp.zeros_like(cnt)

    h = dis_ref[...] * (p_ref[0] + p_ref[1] + u_ref[...]) + b_ref[...]
    ids = lax.broadcasted_iota(jnp.int32, (_B, _BLK), 0)
    onehot_t = jnp.where(ids == batch_ref[...], 1.0, 0.0)
    seg[...] += jnp.dot(onehot_t, h, preferred_element_type=jnp.float32)
    cnt[...] += jnp.sum(onehot_t, axis=1, keepdims=True)

    @pl.when(i == _G - 1)
    def _():
        pooled = seg[...] / jnp.maximum(cnt[...], 1.0)
        out_ref[...] = (jnp.dot(pooled, wl_ref[...],
                                preferred_element_type=jnp.float32)
                        + bl_ref[...])


_tc1_call = pl.pallas_call(
    _tc1_body,
    grid=(_G,),
    in_specs=[
        pl.BlockSpec((_BLK, _D), lambda i: (i, 0)),
        pl.BlockSpec((_D, _HF), lambda i: (0, 0)),
        pl.BlockSpec((_NC, _BLK, _DEGW), lambda i: (0, i, 0)),
    ],
    out_specs=[
        pl.BlockSpec((_BLK, _HF), lambda i: (i, 0)),
        pl.BlockSpec((_BLK, 1), lambda i: (i, 0)),
    ],
    out_shape=[
        jax.ShapeDtypeStruct((_NPAD, _HF), jnp.float32),
        jax.ShapeDtypeStruct((_NPAD, 1), jnp.float32),
    ],
)

_mid_call = pl.pallas_call(
    _mid_body,
    grid=(_G,),
    in_specs=[
        pl.BlockSpec((_NC, _BLK, _HF), lambda i: (0, i, 0)),
        pl.BlockSpec((_BLK, _HF), lambda i: (i, 0)),
        pl.BlockSpec((_BLK, 1), lambda i: (i, 0)),
        pl.BlockSpec((1, _HF), lambda i: (0, 0)),
        pl.BlockSpec((_HF, _HF), lambda i: (0, 0)),
    ],
    out_specs=pl.BlockSpec((_BLK, _HF), lambda i: (i, 0)),
    out_shape=jax.ShapeDtypeStruct((_NPAD, _HF), jnp.float32),
)

_fin_call = pl.pallas_call(
    _fin_body,
    grid=(_G,),
    in_specs=[
        pl.BlockSpec((_NC, _BLK, _HF), lambda i: (0, i, 0)),
        pl.BlockSpec((_BLK, _HF), lambda i: (i, 0)),
        pl.BlockSpec((_BLK, 1), lambda i: (i, 0)),
        pl.BlockSpec((1, _HF), lambda i: (0, 0)),
        pl.BlockSpec((1, _BLK), lambda i: (i, 0)),
        pl.BlockSpec((_HF, _C), lambda i: (0, 0)),
        pl.BlockSpec((1, _C), lambda i: (0, 0)),
    ],
    out_specs=pl.BlockSpec((_B, _C), lambda i: (0, 0)),
    out_shape=jax.ShapeDtypeStruct((_B, _C), jnp.float32),
    scratch_shapes=[
        pltpu.VMEM((_B, _HF), jnp.float32),
        pltpu.VMEM((_B, 1), jnp.float32),
    ],
)


def kernel(x, edge_index, batch, W1, b1, W2, b2, W3, b3, Wl, bl):
    x_p = jnp.pad(x, ((0, _NPAD - _N), (0, 0)))
    src3 = edge_index[0].reshape(_NW, _CH, _K)
    dst3 = edge_index[1].reshape(_NW, _CH, _K)
    batch_p = jnp.pad(batch, (0, _NPAD - _N),
                      constant_values=_B).reshape(_G, _BLK)
    z_h = jnp.zeros((_RPS, _HF), jnp.float32)
    z16 = jnp.zeros((_RPS, _DEGW), jnp.float32)
    ones16 = jnp.ones((_K, _DEGW), jnp.float32)

    degp = _deg_call(dst3, ones16, z16)
    u1, dis = _tc1_call(x_p, W1, degp)
    p1 = _scat_call(u1, src3, dst3, z_h)
    u2 = _mid_call(p1, u1, dis, b1.reshape(1, _HF), W2)
    p2 = _scat_call(u2, src3, dst3, z_h)
    u3 = _mid_call(p2, u2, dis, b2.reshape(1, _HF), W3)
    p3 = _scat_call(u3, src3, dst3, z_h)
    return _fin_call(p3, u3, dis, b3.reshape(1, _HF), batch_p,
                     Wl, bl.reshape(1, _C))


# speedup vs baseline: 17.7947x; 17.7947x over previous
import jax
import jax.numpy as jnp
from jax import lax
from jax.experimental import pallas as pl
from jax.experimental.pallas import tpu as pltpu
from jax.experimental.pallas import tpu_sc as plsc

_N = 10000
_E = 320000
_D = 128
_HF = 128
_C = 16
_B = 64

_NPAD = 10240
_NC = 2
_NS = 16
_NW = _NC * _NS
_EW = _E // _NW
_K = 125
_CH = _EW // _K
_RPS = _NPAD // _NS
_DEGW = 128
_BLK = 1024
_G = _NPAD // _BLK

_sc_mesh = plsc.VectorSubcoreMesh(core_axis_name="c", subcore_axis_name="s")


def _deg_body(dst3, ones_h, z16_h, out, dst_v, ones_v, acc):
    c = lax.axis_index("c")
    s = lax.axis_index("s")
    w = c * _NS + s
    pltpu.sync_copy(z16_h, acc.at[pl.ds(s * _RPS, _RPS)])
    pltpu.sync_copy(dst3.at[w], dst_v)
    pltpu.sync_copy(ones_h, ones_v)
    plsc.subcore_barrier()

    def body(j, carry):
        pltpu.sync_copy(ones_v, acc.at[dst_v.at[j]], add=True)
        return carry

    lax.fori_loop(0, _CH, body, 0)
    plsc.subcore_barrier()
    pltpu.sync_copy(acc.at[pl.ds(s * _RPS, _RPS)],
                    out.at[c].at[pl.ds(s * _RPS, _RPS)])


_deg_call = pl.kernel(
    _deg_body,
    out_type=jax.ShapeDtypeStruct((_NC, _NPAD, _DEGW), jnp.float32),
    mesh=_sc_mesh,
    scratch_types=[
        pltpu.VMEM((_CH, _K), jnp.int32),
        pltpu.VMEM((_K, _DEGW), jnp.float32),
        pltpu.VMEM_SHARED((_NPAD, _DEGW), jnp.float32),
    ],
)


def _scat_body(u_h, src3, dst3, z_h, out, src_v, dst_v, gbuf, acc):
    c = lax.axis_index("c")
    s = lax.axis_index("s")
    w = c * _NS + s
    pltpu.sync_copy(z_h, acc.at[pl.ds(s * _RPS, _RPS)])
    pltpu.sync_copy(src3.at[w], src_v)
    pltpu.sync_copy(dst3.at[w], dst_v)
    plsc.subcore_barrier()

    def body(j, carry):
        pltpu.sync_copy(u_h.at[src_v.at[j]], gbuf)
        pltpu.sync_copy(gbuf, acc.at[dst_v.at[j]], add=True)
        return carry

    lax.fori_loop(0, _CH, body, 0)
    plsc.subcore_barrier()
    pltpu.sync_copy(acc.at[pl.ds(s * _RPS, _RPS)],
                    out.at[c].at[pl.ds(s * _RPS, _RPS)])


_scat_call = pl.kernel(
    _scat_body,
    out_type=jax.ShapeDtypeStruct((_NC, _NPAD, _HF), jnp.float32),
    mesh=_sc_mesh,
    scratch_types=[
        pltpu.VMEM((_CH, _K), jnp.int32),
        pltpu.VMEM((_CH, _K), jnp.int32),
        pltpu.VMEM((_K, _HF), jnp.float32),
        pltpu.VMEM_SHARED((_NPAD, _HF), jnp.float32),
    ],
)


def _tc1_body(x_ref, w1_ref, degp_ref, u1_ref, dis_ref):
    deg = degp_ref[0, :, 0:1] + degp_ref[1, :, 0:1] + 1.0
    dis = lax.rsqrt(deg)
    h = jnp.dot(x_ref[...], w1_ref[...], preferred_element_type=jnp.float32)
    u1_ref[...] = dis * h
    dis_ref[...] = dis


def _mid_body(p_ref, u_ref, dis_ref, b_ref, w_ref, out_ref):
    dis = dis_ref[...]
    t = dis * (p_ref[0] + p_ref[1] + u_ref[...]) + b_ref[...]
    h = jnp.maximum(t, 0.0)
    out_ref[...] = dis * jnp.dot(h, w_ref[...],
                                 preferred_element_type=jnp.float32)


def _fin_body(p_ref, u_ref, dis_ref, b_ref, batch_ref, wl_ref, bl_ref,
              out_ref, seg, cnt):
    i = pl.program_id(0)

    @pl.when(i == 0)
    def _():
        seg[...] = jnp.zeros_like(seg)
        cnt[...] = jnp.zeros_like(cnt)

    h = dis_ref[...] * (p_ref[0] + p_ref[1] + u_ref[...]) + b_ref[...]
    ids = lax.broadcasted_iota(jnp.int32, (_B, _BLK), 0)
    onehot_t = jnp.where(ids == batch_ref[0], 1.0, 0.0)
    seg[...] += jnp.dot(onehot_t, h, preferred_element_type=jnp.float32)
    cnt[...] += jnp.sum(onehot_t, axis=1, keepdims=True)

    @pl.when(i == _G - 1)
    def _():
        pooled = seg[...] / jnp.maximum(cnt[...], 1.0)
        out_ref[...] = (jnp.dot(pooled, wl_ref[...],
                                preferred_element_type=jnp.float32)
                        + bl_ref[...])


_tc1_call = pl.pallas_call(
    _tc1_body,
    grid=(_G,),
    in_specs=[
        pl.BlockSpec((_BLK, _D), lambda i: (i, 0)),
        pl.BlockSpec((_D, _HF), lambda i: (0, 0)),
        pl.BlockSpec((_NC, _BLK, _DEGW), lambda i: (0, i, 0)),
    ],
    out_specs=[
        pl.BlockSpec((_BLK, _HF), lambda i: (i, 0)),
        pl.BlockSpec((_BLK, 1), lambda i: (i, 0)),
    ],
    out_shape=[
        jax.ShapeDtypeStruct((_NPAD, _HF), jnp.float32),
        jax.ShapeDtypeStruct((_NPAD, 1), jnp.float32),
    ],
)

_mid_call = pl.pallas_call(
    _mid_body,
    grid=(_G,),
    in_specs=[
        pl.BlockSpec((_NC, _BLK, _HF), lambda i: (0, i, 0)),
        pl.BlockSpec((_BLK, _HF), lambda i: (i, 0)),
        pl.BlockSpec((_BLK, 1), lambda i: (i, 0)),
        pl.BlockSpec((1, _HF), lambda i: (0, 0)),
        pl.BlockSpec((_HF, _HF), lambda i: (0, 0)),
    ],
    out_specs=pl.BlockSpec((_BLK, _HF), lambda i: (i, 0)),
    out_shape=jax.ShapeDtypeStruct((_NPAD, _HF), jnp.float32),
)

_fin_call = pl.pallas_call(
    _fin_body,
    grid=(_G,),
    in_specs=[
        pl.BlockSpec((_NC, _BLK, _HF), lambda i: (0, i, 0)),
        pl.BlockSpec((_BLK, _HF), lambda i: (i, 0)),
        pl.BlockSpec((_BLK, 1), lambda i: (i, 0)),
        pl.BlockSpec((1, _HF), lambda i: (0, 0)),
        pl.BlockSpec((1, 1, _BLK), lambda i: (i, 0, 0)),
        pl.BlockSpec((_HF, _C), lambda i: (0, 0)),
        pl.BlockSpec((1, _C), lambda i: (0, 0)),
    ],
    out_specs=pl.BlockSpec((_B, _C), lambda i: (0, 0)),
    out_shape=jax.ShapeDtypeStruct((_B, _C), jnp.float32),
    scratch_shapes=[
        pltpu.VMEM((_B, _HF), jnp.float32),
        pltpu.VMEM((_B, 1), jnp.float32),
    ],
)


def kernel(x, edge_index, batch, W1, b1, W2, b2, W3, b3, Wl, bl):
    x_p = jnp.pad(x, ((0, _NPAD - _N), (0, 0)))
    src3 = edge_index[0].reshape(_NW, _CH, _K)
    dst3 = edge_index[1].reshape(_NW, _CH, _K)
    batch_p = jnp.pad(batch, (0, _NPAD - _N),
                      constant_values=_B).reshape(_G, 1, _BLK)
    z_h = jnp.zeros((_RPS, _HF), jnp.float32)
    z16 = jnp.zeros((_RPS, _DEGW), jnp.float32)
    ones16 = jnp.ones((_K, _DEGW), jnp.float32)

    degp = _deg_call(dst3, ones16, z16)
    u1, dis = _tc1_call(x_p, W1, degp)
    p1 = _scat_call(u1, src3, dst3, z_h)
    u2 = _mid_call(p1, u1, dis, b1.reshape(1, _HF), W2)
    p2 = _scat_call(u2, src3, dst3, z_h)
    u3 = _mid_call(p2, u2, dis, b2.reshape(1, _HF), W3)
    p3 = _scat_call(u3, src3, dst3, z_h)
    return _fin_call(p3, u3, dis, b3.reshape(1, _HF), batch_p,
                     Wl, bl.reshape(1, _C))
